# baseline (speedup 1.0000x reference)
import math

import jax
import jax.numpy as jnp
from jax import lax
from jax.experimental import pallas as pl
from jax.experimental.pallas import tpu as pltpu

_INV_SQRT2 = 1.0 / math.sqrt(2.0)
_NUM_ANSWERS = 3129


def _fused_kernel(feat_ref, pos_ref, lang_ref, wvis_ref, wpool_ref, bpool_ref,
                  w1_ref, b1_ref, gamma_ref, beta_ref, w2_ref, b2_ref,
                  out_ref):
    o = feat_ref.shape[1]
    bb = feat_ref.shape[0]
    kp = wvis_ref.shape[0]
    f = feat_ref.shape[2]
    mf = jnp.sum(feat_ref[...], axis=1) * (1.0 / o)
    mp = jnp.sum(pos_ref[...], axis=1) * (1.0 / o)
    langm = jnp.mean(lang_ref[...], axis=1)
    pad = jnp.zeros((bb, kp - f - mp.shape[1]), jnp.float32)
    xcat = jnp.concatenate([mf, mp, pad], axis=1).astype(jnp.bfloat16)
    visn = jnp.dot(xcat, wvis_ref[...], preferred_element_type=jnp.float32)
    x = visn + langm.astype(jnp.bfloat16).astype(jnp.float32)
    pooled = jnp.tanh(
        jnp.dot(x.astype(jnp.bfloat16), wpool_ref[...],
                preferred_element_type=jnp.float32) + bpool_ref[...])
    h = jnp.dot(pooled.astype(jnp.bfloat16), w1_ref[...],
                preferred_element_type=jnp.float32) + b1_ref[...]
    h = h * 0.5 * (1.0 + lax.erf(h * _INV_SQRT2))
    mu = jnp.mean(h, axis=-1, keepdims=True)
    var = jnp.mean((h - mu) ** 2, axis=-1, keepdims=True)
    hn = (h - mu) * lax.rsqrt(var + 1e-12) * gamma_ref[...] + beta_ref[...]
    res = (jnp.dot(hn.astype(jnp.bfloat16), w2_ref[...],
                   preferred_element_type=jnp.float32) + b2_ref[...])
    out_ref[...] = res[:, :out_ref.shape[1]]


def kernel(feat, pos, lang_emb, w_vis, wpool, bpool, w1, b1, gamma, beta,
           w2, b2):
    B = feat.shape[0]
    out = pl.pallas_call(
        _fused_kernel,
        out_shape=jax.ShapeDtypeStruct((B, _NUM_ANSWERS), jnp.float32),
        compiler_params=pltpu.CompilerParams(
            vmem_limit_bytes=60 * 1024 * 1024,
        ),
    )(feat, pos, lang_emb, w_vis, wpool, bpool, w1, b1, gamma, beta, w2, b2)
    return out


# speedup vs baseline: 2.4932x; 2.4932x over previous
import math

import jax
import jax.numpy as jnp
from jax import lax
from jax.experimental import pallas as pl
from jax.experimental.pallas import tpu as pltpu

_INV_SQRT2 = 1.0 / math.sqrt(2.0)
_NUM_ANSWERS = 3129


def _head_kernel(mf_ref, tail_ref, langm_ref, wvis_ref, wpool_ref, bpool_ref,
                 w1_ref, b1_ref, gamma_ref, beta_ref, w2_ref, b2_ref,
                 out_ref, hn_ref):
    k = pl.program_id(0)

    @pl.when(k == 0)
    def _prefix():
        xcat = jnp.concatenate([mf_ref[...], tail_ref[...]], axis=1)
        visn = jnp.dot(xcat, wvis_ref[...],
                       preferred_element_type=jnp.float32)
        x = visn + langm_ref[...].astype(jnp.float32)
        pooled = jnp.tanh(
            jnp.dot(x.astype(jnp.bfloat16), wpool_ref[...],
                    preferred_element_type=jnp.float32) + bpool_ref[...])
        h = jnp.dot(pooled.astype(jnp.bfloat16), w1_ref[...],
                    preferred_element_type=jnp.float32) + b1_ref[...]
        h = h * 0.5 * (1.0 + lax.erf(h * _INV_SQRT2))
        mu = jnp.mean(h, axis=-1, keepdims=True)
        var = jnp.mean((h - mu) ** 2, axis=-1, keepdims=True)
        hn = (h - mu) * lax.rsqrt(var + 1e-12) * gamma_ref[...] + beta_ref[...]
        hn_ref[...] = hn.astype(jnp.bfloat16)

    res = (jnp.dot(hn_ref[...], w2_ref[...],
                   preferred_element_type=jnp.float32) + b2_ref[...])
    out_ref[...] = res[:, :out_ref.shape[1]]


def kernel(feat, pos, lang_emb, w_vis, wpool, bpool, w1, b1, gamma, beta,
           w2, b2):
    B, O, F = feat.shape
    H = wpool.shape[0]
    H2 = w1.shape[1]
    Kp = w_vis.shape[0]
    Ap = w2.shape[1]
    TW = Kp - F

    mf = jnp.mean(feat, axis=1).astype(jnp.bfloat16)
    mp = jnp.mean(pos, axis=1)
    tail = jnp.pad(mp, ((0, 0), (0, TW - 4))).astype(jnp.bfloat16)
    langm = jnp.mean(lang_emb, axis=1).astype(jnp.bfloat16)

    ta = Ap // 2
    out = pl.pallas_call(
        _head_kernel,
        out_shape=jax.ShapeDtypeStruct((B, _NUM_ANSWERS), jnp.float32),
        grid=(2,),
        in_specs=[
            pl.BlockSpec((B, F), lambda j: (0, 0)),
            pl.BlockSpec((B, TW), lambda j: (0, 0)),
            pl.BlockSpec((B, H), lambda j: (0, 0)),
            pl.BlockSpec((Kp, H), lambda j: (0, 0)),
            pl.BlockSpec((H, H), lambda j: (0, 0)),
            pl.BlockSpec((1, H), lambda j: (0, 0)),
            pl.BlockSpec((H, H2), lambda j: (0, 0)),
            pl.BlockSpec((1, H2), lambda j: (0, 0)),
            pl.BlockSpec((1, H2), lambda j: (0, 0)),
            pl.BlockSpec((1, H2), lambda j: (0, 0)),
            pl.BlockSpec((H2, ta), lambda j: (0, j)),
            pl.BlockSpec((1, ta), lambda j: (0, j)),
        ],
        out_specs=pl.BlockSpec((B, ta), lambda j: (0, j)),
        scratch_shapes=[pltpu.VMEM((B, H2), jnp.bfloat16)],
        compiler_params=pltpu.CompilerParams(
            dimension_semantics=("arbitrary",),
            vmem_limit_bytes=48 * 1024 * 1024,
        ),
    )(mf, tail, langm, w_vis, wpool, bpool, w1, b1, gamma, beta, w2, b2)

    return out
